# baseline (speedup 1.0000x reference)
import functools
import math

import jax
import jax.numpy as jnp
from jax import lax
from jax.experimental import pallas as pl
from jax.experimental.pallas import tpu as pltpu
from jax.experimental.pallas import tpu_sc as plsc

N = 10000
E = 320000
HC = 128
NRBF = 64
D3 = 3 * HC

NC = 2
NS = 16
NW = NC * NS
EP_T = E // NW
EP_S = E // NS
NHALF = N // NC
ACC_ROWS = 5120
TRASH = NHALF
GCH = 80
SCH = 224
BIN_PAD = ((EP_S + SCH - 1) // SCH) * SCH
ICH = 2000

_PACK_SH = 13

_INV3 = 1.0 / math.sqrt(3.0)
_INVH = 1.0 / math.sqrt(HC)


def _gather_bin_kernel(tbl_hbm, j_hbm, i_hbm, g_hbm, bpk_hbm, bcnt_hbm):
    c = lax.axis_index("c")
    s = lax.axis_index("s")
    wid = c * NS + s

    def bin_phase(ivm, pb, cbuf):
        def memset(k, _):
            pb[pl.ds(k * 16, 16)] = jnp.full((16,), TRASH, jnp.int32)
            return 0

        lax.fori_loop(0, BIN_PAD // 16, memset, 0)

        lo = c * NHALF

        def chunk(ci, cnt):
            pltpu.sync_copy(i_hbm.at[pl.ds(s * EP_S + ci * ICH, ICH)], ivm)

            def body(k, cnt):
                iv = ivm[pl.ds(k * 16, 16)]
                m = (iv >= lo) & (iv < lo + NHALF)
                dl = iv - lo
                ids = s * EP_S + ci * ICH + k * 16 + lax.iota(jnp.int32, 16)
                w = jnp.bitwise_or(jnp.left_shift(ids, _PACK_SH), dl)
                ps = plsc.cumsum(jnp.where(m, jnp.int32(1), jnp.int32(0)))
                pos = cnt + ps - 1
                plsc.store_scatter(pb, [pos], w, mask=m)
                return cnt + jnp.max(ps)

            return lax.fori_loop(0, ICH // 16, body, cnt)

        cnt = lax.fori_loop(0, EP_S // ICH, chunk, jnp.int32(0))

        pltpu.sync_copy(pb, bpk_hbm.at[wid])
        cbuf[...] = jnp.where(lax.iota(jnp.int32, 16) == 0,
                              jnp.full((16,), cnt, jnp.int32),
                              jnp.zeros((16,), jnp.int32))
        pltpu.sync_copy(cbuf, bcnt_hbm.at[wid])

    pl.run_scoped(bin_phase,
                  pltpu.VMEM((ICH,), jnp.int32),
                  pltpu.VMEM((BIN_PAD,), jnp.int32),
                  pltpu.VMEM((16,), jnp.int32))

    def gather_phase(jvm, g0, g1, s0, s1):
        pltpu.sync_copy(j_hbm.at[pl.ds(wid * EP_T, EP_T)], jvm)
        gs, ss = (g0, g1), (s0, s1)

        def issue(t, b):
            idxs = jvm.at[pl.ds(t * GCH, GCH)]
            pltpu.async_copy(tbl_hbm.at[idxs], gs[b], ss[b])

        def finish(t, b):
            base = wid * EP_T + t * GCH
            idxs = jvm.at[pl.ds(t * GCH, GCH)]
            pltpu.make_async_copy(tbl_hbm.at[idxs], gs[b], ss[b]).wait()
            pltpu.sync_copy(gs[b], g_hbm.at[pl.ds(base, GCH)])

        nch = EP_T // GCH

        def body(t, _):
            @pl.when(t % 2 == 1)
            def _():
                issue(t, 1)
                finish(t - 1, 0)

            @pl.when(t % 2 == 0)
            def _():
                issue(t, 0)
                finish(t - 1, 1)

            return 0

        issue(0, 0)
        lax.fori_loop(1, nch, body, 0)
        finish(nch - 1, (nch - 1) % 2)

    pl.run_scoped(gather_phase,
                  pltpu.VMEM((EP_T,), jnp.int32),
                  pltpu.VMEM((GCH, (HC + D3) // 2), jnp.int32),
                  pltpu.VMEM((GCH, (HC + D3) // 2), jnp.int32),
                  pltpu.SemaphoreType.DMA,
                  pltpu.SemaphoreType.DMA)


def _edge_mlp_kernel(g_ref, rbf_ref, ev0_ref, ev1_ref, ev2_ref,
                     W1_ref, b1_ref, W2_ref, b2_ref, Wr_ref, br_ref,
                     mv0_ref, mv1_ref, mv2_ref, mx_ref):
    dn = (((1,), (1,)), ((), ()))
    g = g_ref[...].astype(jnp.float32)
    gx = g[:, :HC]
    h = lax.dot_general(gx, W1_ref[...], dn, preferred_element_type=jnp.float32)
    h = h + b1_ref[...]
    h = (h * jax.nn.sigmoid(h)) * (1.0 / 0.6)
    xh = lax.dot_general(h, W2_ref[...], dn, preferred_element_type=jnp.float32)
    xh = xh + b2_ref[...]
    rh = lax.dot_general(rbf_ref[...], Wr_ref[...], dn,
                         preferred_element_type=jnp.float32)
    rh = rh + br_ref[...]
    t = xh * rh * _INV3
    t1 = t[:, :HC]
    t2 = t[:, HC:2 * HC]
    mx_ref[...] = t[:, 2 * HC:]
    evs = (ev0_ref[...], ev1_ref[...], ev2_ref[...])
    mv_refs = (mv0_ref, mv1_ref, mv2_ref)
    for d in range(3):
        mv_refs[d][...] = (
            t1 * g[:, (d + 1) * HC:(d + 2) * HC] + t2 * evs[d]) * _INVH


def _scatter_kernel(m0_hbm, m1_hbm, m2_hbm, m3_hbm, bpk_hbm, bcnt_hbm,
                    o0_hbm, o1_hbm, o2_hbm, o3_hbm, acc):
    c = lax.axis_index("c")
    s = lax.axis_index("s")
    wid = c * NS + s

    def outer(pbv, ic0, ic1, dc0, dc1, cbuf, rows0, rows1, zb, sem0, sem1):
        pltpu.sync_copy(bcnt_hbm.at[wid], cbuf)
        cnt = jnp.max(cbuf[...])
        nch = jnp.maximum((cnt + (SCH - 1)) // SCH, 1)
        pltpu.sync_copy(bpk_hbm.at[wid], pbv)

        for r in range(16):
            for q in range(HC // 16):
                zb[r, pl.ds(q * 16, 16)] = jnp.zeros((16,), jnp.float32)

        rows_per_tile = ACC_ROWS // NS
        ics, dcs, rowss, sems = (ic0, ic1), (dc0, dc1), (rows0, rows1), \
            (sem0, sem1)

        for src_hbm, out_hbm in ((m0_hbm, o0_hbm), (m1_hbm, o1_hbm),
                                 (m2_hbm, o2_hbm), (m3_hbm, o3_hbm)):
            def zbody(t, _):
                pltpu.sync_copy(zb,
                                acc.at[pl.ds(s * rows_per_tile + t * 16, 16)])
                return 0

            lax.fori_loop(0, rows_per_tile // 16, zbody, 0)
            plsc.subcore_barrier()

            def issue(k, b):
                for q in range(SCH // 16):
                    w = pbv[pl.ds(k * SCH + q * 16, 16)]
                    ics[b][pl.ds(q * 16, 16)] = lax.shift_right_logical(
                        w, jnp.full((16,), _PACK_SH, jnp.int32))
                    dcs[b][pl.ds(q * 16, 16)] = jnp.bitwise_and(
                        w, (1 << _PACK_SH) - 1)
                pltpu.async_copy(src_hbm.at[ics[b]], rowss[b], sems[b])

            def finish(b):
                pltpu.make_async_copy(src_hbm.at[ics[b]], rowss[b],
                                      sems[b]).wait()
                pltpu.sync_copy(rowss[b], acc.at[dcs[b]], add=True)

            def body(k, _):
                @pl.when(k % 2 == 1)
                def _():
                    issue(k, 1)
                    finish(0)

                @pl.when(k % 2 == 0)
                def _():
                    issue(k, 0)
                    finish(1)

                return 0

            issue(0, 0)
            lax.fori_loop(1, nch, body, 0)

            @pl.when(nch % 2 == 1)
            def _():
                finish(0)

            @pl.when(nch % 2 == 0)
            def _():
                finish(1)

            plsc.subcore_barrier()

            @pl.when(s < 15)
            def _():
                pltpu.sync_copy(acc.at[pl.ds(s * 312, 312)],
                                out_hbm.at[pl.ds(c * NHALF + s * 312, 312)])

            @pl.when(s == 15)
            def _():
                pltpu.sync_copy(acc.at[pl.ds(4680, 320)],
                                out_hbm.at[pl.ds(c * NHALF + 4680, 320)])

            plsc.subcore_barrier()

    pl.run_scoped(outer,
                  pltpu.VMEM((BIN_PAD,), jnp.int32),
                  pltpu.VMEM((SCH,), jnp.int32),
                  pltpu.VMEM((SCH,), jnp.int32),
                  pltpu.VMEM((SCH,), jnp.int32),
                  pltpu.VMEM((SCH,), jnp.int32),
                  pltpu.VMEM((16,), jnp.int32),
                  pltpu.VMEM((SCH, HC), jnp.float32),
                  pltpu.VMEM((SCH, HC), jnp.float32),
                  pltpu.VMEM((16, HC), jnp.float32),
                  pltpu.SemaphoreType.DMA,
                  pltpu.SemaphoreType.DMA)


_SC_MESH = plsc.VectorSubcoreMesh(core_axis_name="c", subcore_axis_name="s")
_SC_PARAMS = pltpu.CompilerParams(needs_layout_passes=False)

_gather_call = functools.partial(
    pl.kernel, mesh=_SC_MESH, compiler_params=_SC_PARAMS,
    out_type=[
        jax.ShapeDtypeStruct((E, (HC + D3) // 2), jnp.int32),
        jax.ShapeDtypeStruct((NW, BIN_PAD), jnp.int32),
        jax.ShapeDtypeStruct((NW, 16), jnp.int32),
    ])(_gather_bin_kernel)

_scatter_call = functools.partial(
    pl.kernel, mesh=_SC_MESH, compiler_params=_SC_PARAMS,
    out_type=[jax.ShapeDtypeStruct((N, HC), jnp.float32)] * 4,
    scratch_types=[pltpu.VMEM_SHARED((ACC_ROWS, HC), jnp.float32)],
    )(_scatter_kernel)

_EB = 2000


def _edge_mlp(g, rbf, ev0, ev1, ev2, W1, b1, W2, b2, Wr, br):
    grid = (E // _EB,)
    row = lambda m: (m, 0)
    fixed = lambda m: (0, 0)
    return pl.pallas_call(
        _edge_mlp_kernel,
        grid=grid,
        in_specs=[
            pl.BlockSpec((_EB, HC + D3), row),
            pl.BlockSpec((_EB, NRBF), row),
            pl.BlockSpec((_EB, 1), row),
            pl.BlockSpec((_EB, 1), row),
            pl.BlockSpec((_EB, 1), row),
            pl.BlockSpec((HC // 2, HC), fixed),
            pl.BlockSpec((1, HC // 2), fixed),
            pl.BlockSpec((D3, HC // 2), fixed),
            pl.BlockSpec((1, D3), fixed),
            pl.BlockSpec((D3, NRBF), fixed),
            pl.BlockSpec((1, D3), fixed),
        ],
        out_specs=[pl.BlockSpec((_EB, HC), row)] * 4,
        out_shape=[jax.ShapeDtypeStruct((E, HC), jnp.float32)] * 4,
    )(g, rbf, ev0, ev1, ev2, W1, b1, W2, b2, Wr, br)


def kernel(x, vec, edge_rbf, edge_vector, W1, b1, W2, b2, Wr, br, edge_index):
    vec2 = vec.reshape(N, D3)
    tbl_bf = jnp.concatenate([x, vec2], axis=1).astype(jnp.bfloat16)
    tbl = lax.bitcast_convert_type(
        tbl_bf.reshape(N, (HC + D3) // 2, 2), jnp.int32)
    jj = edge_index[0]
    ii = edge_index[1]
    g, bpk, bcnt = _gather_call(tbl, jj, ii)
    g_bf = lax.bitcast_convert_type(g, jnp.bfloat16).reshape(E, HC + D3)
    mv0, mv1, mv2, mx = _edge_mlp(g_bf, edge_rbf,
                                  edge_vector[:, 0:1], edge_vector[:, 1:2],
                                  edge_vector[:, 2:3],
                                  W1, b1.reshape(1, -1), W2, b2.reshape(1, -1),
                                  Wr, br.reshape(1, -1))
    dv0, dv1, dv2c, dx = _scatter_call(mv0, mv1, mv2, mx, bpk, bcnt)
    d_vec = jnp.stack([dv0, dv1, dv2c], axis=1)
    return (dx, d_vec)


# speedup vs baseline: 2.3531x; 2.3531x over previous
import functools
import math

import jax
import jax.numpy as jnp
from jax import lax
from jax.experimental import pallas as pl
from jax.experimental.pallas import tpu as pltpu
from jax.experimental.pallas import tpu_sc as plsc

N = 10000
E = 320000
HC = 128
NRBF = 64
D3 = 3 * HC

NC = 2
NS = 16
NW = NC * NS
EP_T = E // NW
EP_S = E // NS
NHALF = N // NC
ACC_ROWS = 5120
TRASH = NHALF
GCH = 80
SCH = 224
BIN_PAD = ((EP_S + SCH - 1) // SCH) * SCH
ICH = 2000

_PACK_SH = 13

_INV3 = 1.0 / math.sqrt(3.0)
_INVH = 1.0 / math.sqrt(HC)


def _gather_bin_kernel(tbl_hbm, j_hbm, i_hbm, g_hbm, bpk_hbm, bcnt_hbm):
    c = lax.axis_index("c")
    s = lax.axis_index("s")
    wid = c * NS + s

    def bin_phase(ivm, pb, cbuf):
        def memset(k, _):
            pb[pl.ds(k * 16, 16)] = jnp.full((16,), TRASH, jnp.int32)
            return 0

        lax.fori_loop(0, BIN_PAD // 16, memset, 0)

        lo = c * NHALF

        def chunk(ci, cnt):
            pltpu.sync_copy(i_hbm.at[pl.ds(s * EP_S + ci * ICH, ICH)], ivm)

            def body(k, cnt):
                iv = ivm[pl.ds(k * 16, 16)]
                m = (iv >= lo) & (iv < lo + NHALF)
                dl = iv - lo
                ids = s * EP_S + ci * ICH + k * 16 + lax.iota(jnp.int32, 16)
                w = jnp.bitwise_or(jnp.left_shift(ids, _PACK_SH), dl)
                ps = plsc.cumsum(jnp.where(m, jnp.int32(1), jnp.int32(0)))
                pos = cnt + ps - 1
                plsc.store_scatter(pb, [pos], w, mask=m)
                return cnt + jnp.max(ps)

            return lax.fori_loop(0, ICH // 16, body, cnt)

        cnt = lax.fori_loop(0, EP_S // ICH, chunk, jnp.int32(0))

        pltpu.sync_copy(pb, bpk_hbm.at[wid])
        cbuf[...] = jnp.where(lax.iota(jnp.int32, 16) == 0,
                              jnp.full((16,), cnt, jnp.int32),
                              jnp.zeros((16,), jnp.int32))
        pltpu.sync_copy(cbuf, bcnt_hbm.at[wid])

    pl.run_scoped(bin_phase,
                  pltpu.VMEM((ICH,), jnp.int32),
                  pltpu.VMEM((BIN_PAD,), jnp.int32),
                  pltpu.VMEM((16,), jnp.int32))

    def gather_phase(jvm, g0, g1, s0, s1):
        pltpu.sync_copy(j_hbm.at[pl.ds(wid * EP_T, EP_T)], jvm)
        gs, ss = (g0, g1), (s0, s1)

        def issue(t, b):
            idxs = jvm.at[pl.ds(t * GCH, GCH)]
            pltpu.async_copy(tbl_hbm.at[idxs], gs[b], ss[b])

        def finish(t, b):
            base = wid * EP_T + t * GCH
            idxs = jvm.at[pl.ds(t * GCH, GCH)]
            pltpu.make_async_copy(tbl_hbm.at[idxs], gs[b], ss[b]).wait()
            pltpu.sync_copy(gs[b], g_hbm.at[pl.ds(base, GCH)])

        nch = EP_T // GCH

        def body(t, _):
            @pl.when(t % 2 == 1)
            def _():
                issue(t, 1)
                finish(t - 1, 0)

            @pl.when(t % 2 == 0)
            def _():
                issue(t, 0)
                finish(t - 1, 1)

            return 0

        issue(0, 0)
        lax.fori_loop(1, nch, body, 0)
        finish(nch - 1, (nch - 1) % 2)

    pl.run_scoped(gather_phase,
                  pltpu.VMEM((EP_T,), jnp.int32),
                  pltpu.VMEM((GCH, (HC + D3) // 2), jnp.int32),
                  pltpu.VMEM((GCH, (HC + D3) // 2), jnp.int32),
                  pltpu.SemaphoreType.DMA,
                  pltpu.SemaphoreType.DMA)


def _pack_kernel(x_ref, v_ref, o_ref):
    lo = jnp.concatenate([x_ref[...], v_ref[:, :HC]], axis=1)
    hi = v_ref[:, HC:]
    lo_b = lax.bitcast_convert_type(
        lo.astype(jnp.bfloat16).astype(jnp.float32), jnp.int32)
    hi_b = lax.bitcast_convert_type(
        hi.astype(jnp.bfloat16).astype(jnp.float32), jnp.int32)
    o_ref[...] = jnp.bitwise_or(
        jnp.bitwise_and(jnp.right_shift(lo_b, 16), jnp.int32(0xFFFF)),
        jnp.bitwise_and(hi_b, jnp.int32(-65536)))


_NB = 2000


def _pack(x, vec2):
    row = lambda m: (m, 0)
    return pl.pallas_call(
        _pack_kernel,
        grid=(N // _NB,),
        in_specs=[
            pl.BlockSpec((_NB, HC), row),
            pl.BlockSpec((_NB, D3), row),
        ],
        out_specs=pl.BlockSpec((_NB, (HC + D3) // 2), row),
        out_shape=jax.ShapeDtypeStruct((N, (HC + D3) // 2), jnp.int32),
    )(x, vec2)


def _edge_mlp_kernel(g_ref, rbf_ref, ev0_ref, ev1_ref, ev2_ref,
                     W1_ref, b1_ref, W2_ref, b2_ref, Wr_ref, br_ref,
                     mv0_ref, mv1_ref, mv2_ref, mx_ref):
    dn = (((1,), (1,)), ((), ()))
    w = g_ref[...]
    f_lo = lax.bitcast_convert_type(jnp.left_shift(w, 16), jnp.float32)
    f_hi = lax.bitcast_convert_type(jnp.bitwise_and(w, jnp.int32(-65536)),
                                    jnp.float32)
    gx = f_lo[:, :HC]
    gslab = (f_lo[:, HC:], f_hi[:, :HC], f_hi[:, HC:])
    h = lax.dot_general(gx, W1_ref[...], dn, preferred_element_type=jnp.float32)
    h = h + b1_ref[...]
    h = (h * jax.nn.sigmoid(h)) * (1.0 / 0.6)
    xh = lax.dot_general(h, W2_ref[...], dn, preferred_element_type=jnp.float32)
    xh = xh + b2_ref[...]
    rh = lax.dot_general(rbf_ref[...], Wr_ref[...], dn,
                         preferred_element_type=jnp.float32)
    rh = rh + br_ref[...]
    t = xh * rh * _INV3
    t1 = t[:, :HC]
    t2 = t[:, HC:2 * HC]
    mx_ref[...] = t[:, 2 * HC:]
    evs = (ev0_ref[...], ev1_ref[...], ev2_ref[...])
    mv_refs = (mv0_ref, mv1_ref, mv2_ref)
    for d in range(3):
        mv_refs[d][...] = (t1 * gslab[d] + t2 * evs[d]) * _INVH


def _scatter_kernel(m0_hbm, m1_hbm, m2_hbm, m3_hbm, bpk_hbm, bcnt_hbm,
                    o0_hbm, o1_hbm, o2_hbm, o3_hbm, acc):
    c = lax.axis_index("c")
    s = lax.axis_index("s")
    wid = c * NS + s

    def outer(pbv, ic0, ic1, dc0, dc1, cbuf, rows0, rows1, zb, sem0, sem1):
        pltpu.sync_copy(bcnt_hbm.at[wid], cbuf)
        cnt = jnp.max(cbuf[...])
        nch = jnp.maximum((cnt + (SCH - 1)) // SCH, 1)
        pltpu.sync_copy(bpk_hbm.at[wid], pbv)

        for r in range(16):
            for q in range(HC // 16):
                zb[r, pl.ds(q * 16, 16)] = jnp.zeros((16,), jnp.float32)

        rows_per_tile = ACC_ROWS // NS
        ics, dcs, rowss, sems = (ic0, ic1), (dc0, dc1), (rows0, rows1), \
            (sem0, sem1)

        for src_hbm, out_hbm in ((m0_hbm, o0_hbm), (m1_hbm, o1_hbm),
                                 (m2_hbm, o2_hbm), (m3_hbm, o3_hbm)):
            def zbody(t, _):
                pltpu.sync_copy(zb,
                                acc.at[pl.ds(s * rows_per_tile + t * 16, 16)])
                return 0

            lax.fori_loop(0, rows_per_tile // 16, zbody, 0)
            plsc.subcore_barrier()

            def issue(k, b):
                for q in range(SCH // 16):
                    w = pbv[pl.ds(k * SCH + q * 16, 16)]
                    ics[b][pl.ds(q * 16, 16)] = lax.shift_right_logical(
                        w, jnp.full((16,), _PACK_SH, jnp.int32))
                    dcs[b][pl.ds(q * 16, 16)] = jnp.bitwise_and(
                        w, (1 << _PACK_SH) - 1)
                pltpu.async_copy(src_hbm.at[ics[b]], rowss[b], sems[b])

            def finish(b):
                pltpu.make_async_copy(src_hbm.at[ics[b]], rowss[b],
                                      sems[b]).wait()
                pltpu.sync_copy(rowss[b], acc.at[dcs[b]], add=True)

            def body(k, _):
                @pl.when(k % 2 == 1)
                def _():
                    issue(k, 1)
                    finish(0)

                @pl.when(k % 2 == 0)
                def _():
                    issue(k, 0)
                    finish(1)

                return 0

            issue(0, 0)
            lax.fori_loop(1, nch, body, 0)

            @pl.when(nch % 2 == 1)
            def _():
                finish(0)

            @pl.when(nch % 2 == 0)
            def _():
                finish(1)

            plsc.subcore_barrier()

            @pl.when(s < 15)
            def _():
                pltpu.sync_copy(acc.at[pl.ds(s * 312, 312)],
                                out_hbm.at[pl.ds(c * NHALF + s * 312, 312)])

            @pl.when(s == 15)
            def _():
                pltpu.sync_copy(acc.at[pl.ds(4680, 320)],
                                out_hbm.at[pl.ds(c * NHALF + 4680, 320)])

            plsc.subcore_barrier()

    pl.run_scoped(outer,
                  pltpu.VMEM((BIN_PAD,), jnp.int32),
                  pltpu.VMEM((SCH,), jnp.int32),
                  pltpu.VMEM((SCH,), jnp.int32),
                  pltpu.VMEM((SCH,), jnp.int32),
                  pltpu.VMEM((SCH,), jnp.int32),
                  pltpu.VMEM((16,), jnp.int32),
                  pltpu.VMEM((SCH, HC), jnp.float32),
                  pltpu.VMEM((SCH, HC), jnp.float32),
                  pltpu.VMEM((16, HC), jnp.float32),
                  pltpu.SemaphoreType.DMA,
                  pltpu.SemaphoreType.DMA)


_SC_MESH = plsc.VectorSubcoreMesh(core_axis_name="c", subcore_axis_name="s")
_SC_PARAMS = pltpu.CompilerParams(needs_layout_passes=False)

_gather_call = functools.partial(
    pl.kernel, mesh=_SC_MESH, compiler_params=_SC_PARAMS,
    out_type=[
        jax.ShapeDtypeStruct((E, (HC + D3) // 2), jnp.int32),
        jax.ShapeDtypeStruct((NW, BIN_PAD), jnp.int32),
        jax.ShapeDtypeStruct((NW, 16), jnp.int32),
    ])(_gather_bin_kernel)

_scatter_call = functools.partial(
    pl.kernel, mesh=_SC_MESH, compiler_params=_SC_PARAMS,
    out_type=[jax.ShapeDtypeStruct((N, HC), jnp.float32)] * 4,
    scratch_types=[pltpu.VMEM_SHARED((ACC_ROWS, HC), jnp.float32)],
    )(_scatter_kernel)

_EB = 2000


def _edge_mlp(g, rbf, ev0, ev1, ev2, W1, b1, W2, b2, Wr, br):
    grid = (E // _EB,)
    row = lambda m: (m, 0)
    fixed = lambda m: (0, 0)
    return pl.pallas_call(
        _edge_mlp_kernel,
        grid=grid,
        in_specs=[
            pl.BlockSpec((_EB, (HC + D3) // 2), row),
            pl.BlockSpec((_EB, NRBF), row),
            pl.BlockSpec((_EB, 1), row),
            pl.BlockSpec((_EB, 1), row),
            pl.BlockSpec((_EB, 1), row),
            pl.BlockSpec((HC // 2, HC), fixed),
            pl.BlockSpec((1, HC // 2), fixed),
            pl.BlockSpec((D3, HC // 2), fixed),
            pl.BlockSpec((1, D3), fixed),
            pl.BlockSpec((D3, NRBF), fixed),
            pl.BlockSpec((1, D3), fixed),
        ],
        out_specs=[pl.BlockSpec((_EB, HC), row)] * 4,
        out_shape=[jax.ShapeDtypeStruct((E, HC), jnp.float32)] * 4,
    )(g, rbf, ev0, ev1, ev2, W1, b1, W2, b2, Wr, br)


def kernel(x, vec, edge_rbf, edge_vector, W1, b1, W2, b2, Wr, br, edge_index):
    vec2 = vec.reshape(N, D3)
    tbl = _pack(x, vec2)
    jj = edge_index[0]
    ii = edge_index[1]
    g, bpk, bcnt = _gather_call(tbl, jj, ii)
    mv0, mv1, mv2, mx = _edge_mlp(g, edge_rbf,
                                  edge_vector[:, 0:1], edge_vector[:, 1:2],
                                  edge_vector[:, 2:3],
                                  W1, b1.reshape(1, -1), W2, b2.reshape(1, -1),
                                  Wr, br.reshape(1, -1))
    dv0, dv1, dv2c, dx = _scatter_call(mv0, mv1, mv2, mx, bpk, bcnt)
    d_vec = jnp.stack([dv0, dv1, dv2c], axis=1)
    return (dx, d_vec)


# speedup vs baseline: 2.3538x; 1.0003x over previous
import functools
import math

import jax
import jax.numpy as jnp
from jax import lax
from jax.experimental import pallas as pl
from jax.experimental.pallas import tpu as pltpu
from jax.experimental.pallas import tpu_sc as plsc

N = 10000
E = 320000
HC = 128
NRBF = 64
D3 = 3 * HC

NC = 2
NS = 16
NW = NC * NS
EP_T = E // NW
EP_S = E // NS
NHALF = N // NC
ACC_ROWS = 5120
TRASH = NHALF
GCH = 80
SCH = 224
BIN_PAD = ((EP_S + SCH - 1) // SCH) * SCH
ICH = 2000

_PACK_SH = 13

_INV3 = 1.0 / math.sqrt(3.0)
_INVH = 1.0 / math.sqrt(HC)


def _gather_bin_kernel(tbl_hbm, j_hbm, i_hbm, g_hbm, bpk_hbm, bcnt_hbm):
    c = lax.axis_index("c")
    s = lax.axis_index("s")
    wid = c * NS + s

    def bin_phase(ivm, pb, cbuf):
        def memset(k, _):
            pb[pl.ds(k * 16, 16)] = jnp.full((16,), TRASH, jnp.int32)
            return 0

        lax.fori_loop(0, BIN_PAD // 16, memset, 0)

        lo = c * NHALF

        def chunk(ci, cnt):
            pltpu.sync_copy(i_hbm.at[pl.ds(s * EP_S + ci * ICH, ICH)], ivm)

            def body(k, cnt):
                iv = ivm[pl.ds(k * 16, 16)]
                m = (iv >= lo) & (iv < lo + NHALF)
                dl = iv - lo
                ids = s * EP_S + ci * ICH + k * 16 + lax.iota(jnp.int32, 16)
                w = jnp.bitwise_or(jnp.left_shift(ids, _PACK_SH), dl)
                ps = plsc.cumsum(jnp.where(m, jnp.int32(1), jnp.int32(0)))
                pos = cnt + ps - 1
                plsc.store_scatter(pb, [pos], w, mask=m)
                return cnt + jnp.max(ps)

            return lax.fori_loop(0, ICH // 16, body, cnt)

        cnt = lax.fori_loop(0, EP_S // ICH, chunk, jnp.int32(0))

        pltpu.sync_copy(pb, bpk_hbm.at[wid])
        cbuf[...] = jnp.where(lax.iota(jnp.int32, 16) == 0,
                              jnp.full((16,), cnt, jnp.int32),
                              jnp.zeros((16,), jnp.int32))
        pltpu.sync_copy(cbuf, bcnt_hbm.at[wid])

    pl.run_scoped(bin_phase,
                  pltpu.VMEM((ICH,), jnp.int32),
                  pltpu.VMEM((BIN_PAD,), jnp.int32),
                  pltpu.VMEM((16,), jnp.int32))

    def gather_phase(jvm, g0, g1, s0, s1):
        pltpu.sync_copy(j_hbm.at[pl.ds(wid * EP_T, EP_T)], jvm)
        gs, ss = (g0, g1), (s0, s1)

        def issue(t, b):
            idxs = jvm.at[pl.ds(t * GCH, GCH)]
            pltpu.async_copy(tbl_hbm.at[idxs], gs[b], ss[b])

        def finish(t, b):
            base = wid * EP_T + t * GCH
            idxs = jvm.at[pl.ds(t * GCH, GCH)]
            pltpu.make_async_copy(tbl_hbm.at[idxs], gs[b], ss[b]).wait()
            pltpu.sync_copy(gs[b], g_hbm.at[pl.ds(base, GCH)])

        nch = EP_T // GCH

        def body(t, _):
            @pl.when(t % 2 == 1)
            def _():
                issue(t, 1)
                finish(t - 1, 0)

            @pl.when(t % 2 == 0)
            def _():
                issue(t, 0)
                finish(t - 1, 1)

            return 0

        issue(0, 0)
        lax.fori_loop(1, nch, body, 0)
        finish(nch - 1, (nch - 1) % 2)

    pl.run_scoped(gather_phase,
                  pltpu.VMEM((EP_T,), jnp.int32),
                  pltpu.VMEM((GCH, (HC + D3) // 2), jnp.int32),
                  pltpu.VMEM((GCH, (HC + D3) // 2), jnp.int32),
                  pltpu.SemaphoreType.DMA,
                  pltpu.SemaphoreType.DMA)


def _pack_kernel(x_ref, v_ref, o_ref):
    lo = jnp.concatenate([x_ref[...], v_ref[:, :HC]], axis=1)
    hi = v_ref[:, HC:]
    lo_b = lax.bitcast_convert_type(
        lo.astype(jnp.bfloat16).astype(jnp.float32), jnp.int32)
    hi_b = lax.bitcast_convert_type(
        hi.astype(jnp.bfloat16).astype(jnp.float32), jnp.int32)
    o_ref[...] = jnp.bitwise_or(
        jnp.bitwise_and(jnp.right_shift(lo_b, 16), jnp.int32(0xFFFF)),
        jnp.bitwise_and(hi_b, jnp.int32(-65536)))


_NB = 2000


def _pack(x, vec2):
    row = lambda m: (m, 0)
    return pl.pallas_call(
        _pack_kernel,
        grid=(N // _NB,),
        in_specs=[
            pl.BlockSpec((_NB, HC), row),
            pl.BlockSpec((_NB, D3), row),
        ],
        out_specs=pl.BlockSpec((_NB, (HC + D3) // 2), row),
        out_shape=jax.ShapeDtypeStruct((N, (HC + D3) // 2), jnp.int32),
    )(x, vec2)


def _edge_mlp_kernel(g_ref, rbf_ref, ev0_ref, ev1_ref, ev2_ref,
                     W1_ref, b1_ref, W2_ref, b2_ref, Wr_ref, br_ref,
                     mv0_ref, mv1_ref, mv2_ref, mx_ref):
    dn = (((1,), (1,)), ((), ()))
    w = g_ref[...]
    f_lo = lax.bitcast_convert_type(jnp.left_shift(w, 16), jnp.float32)
    f_hi = lax.bitcast_convert_type(jnp.bitwise_and(w, jnp.int32(-65536)),
                                    jnp.float32)
    gx = f_lo[:, :HC]
    gslab = (f_lo[:, HC:], f_hi[:, :HC], f_hi[:, HC:])
    h = lax.dot_general(gx, W1_ref[...], dn, preferred_element_type=jnp.float32)
    h = h + b1_ref[...]
    h = (h * jax.nn.sigmoid(h)) * (1.0 / 0.6)
    xh = lax.dot_general(h, W2_ref[...], dn, preferred_element_type=jnp.float32)
    xh = xh + b2_ref[...]
    rh = lax.dot_general(rbf_ref[...], Wr_ref[...], dn,
                         preferred_element_type=jnp.float32)
    rh = rh + br_ref[...]
    t = xh * rh * _INV3
    t1 = t[:, :HC]
    t2 = t[:, HC:2 * HC]
    mx_ref[...] = t[:, 2 * HC:]
    evs = (ev0_ref[...], ev1_ref[...], ev2_ref[...])
    mv_refs = (mv0_ref, mv1_ref, mv2_ref)
    for d in range(3):
        mv_refs[d][...] = (t1 * gslab[d] + t2 * evs[d]) * _INVH


def _scatter_kernel(m0_hbm, m1_hbm, m2_hbm, m3_hbm, bpk_hbm, bcnt_hbm,
                    o0_hbm, o1_hbm, o2_hbm, o3_hbm, acc):
    c = lax.axis_index("c")
    s = lax.axis_index("s")
    wid = c * NS + s

    def outer(pbv, ic0, ic1, dc0, dc1, cbuf, rows0, rows1, zb,
              sem0, sem1, sa0, sa1):
        pltpu.sync_copy(bcnt_hbm.at[wid], cbuf)
        cnt = jnp.max(cbuf[...])
        nch = jnp.maximum((cnt + (SCH - 1)) // SCH, 1)
        pltpu.sync_copy(bpk_hbm.at[wid], pbv)

        for r in range(16):
            for q in range(HC // 16):
                zb[r, pl.ds(q * 16, 16)] = jnp.zeros((16,), jnp.float32)

        rows_per_tile = ACC_ROWS // NS
        ics, dcs, rowss, sems = (ic0, ic1), (dc0, dc1), (rows0, rows1), \
            (sem0, sem1)
        sads = (sa0, sa1)

        for src_hbm, out_hbm in ((m0_hbm, o0_hbm), (m1_hbm, o1_hbm),
                                 (m2_hbm, o2_hbm), (m3_hbm, o3_hbm)):
            def zbody(t, _):
                pltpu.sync_copy(zb,
                                acc.at[pl.ds(s * rows_per_tile + t * 16, 16)])
                return 0

            lax.fori_loop(0, rows_per_tile // 16, zbody, 0)
            plsc.subcore_barrier()

            def issue(k, b):
                for q in range(SCH // 16):
                    w = pbv[pl.ds(k * SCH + q * 16, 16)]
                    ics[b][pl.ds(q * 16, 16)] = lax.shift_right_logical(
                        w, jnp.full((16,), _PACK_SH, jnp.int32))
                    dcs[b][pl.ds(q * 16, 16)] = jnp.bitwise_and(
                        w, (1 << _PACK_SH) - 1)
                pltpu.async_copy(src_hbm.at[ics[b]], rowss[b], sems[b])

            def wait_gather(b):
                pltpu.make_async_copy(src_hbm.at[ics[b]], rowss[b],
                                      sems[b]).wait()

            def start_add(b):
                pltpu.async_copy(rowss[b], acc.at[dcs[b]], sads[b], add=True)

            def wait_add(b):
                pltpu.make_async_copy(rowss[b], acc.at[dcs[b]],
                                      sads[b]).wait()

            def step(k, b):
                @pl.when(k >= 2)
                def _():
                    wait_add(b)

                issue(k, b)
                wait_gather(1 - b)
                start_add(1 - b)

            def body(k, _):
                @pl.when(k % 2 == 1)
                def _():
                    step(k, 1)

                @pl.when(k % 2 == 0)
                def _():
                    step(k, 0)

                return 0

            issue(0, 0)
            lax.fori_loop(1, nch, body, 0)

            def tail(b):
                wait_gather(b)
                start_add(b)

                @pl.when(nch >= 2)
                def _():
                    wait_add(1 - b)

                wait_add(b)

            @pl.when(nch % 2 == 1)
            def _():
                tail(0)

            @pl.when(nch % 2 == 0)
            def _():
                tail(1)

            plsc.subcore_barrier()

            @pl.when(s < 15)
            def _():
                pltpu.sync_copy(acc.at[pl.ds(s * 312, 312)],
                                out_hbm.at[pl.ds(c * NHALF + s * 312, 312)])

            @pl.when(s == 15)
            def _():
                pltpu.sync_copy(acc.at[pl.ds(4680, 320)],
                                out_hbm.at[pl.ds(c * NHALF + 4680, 320)])

            plsc.subcore_barrier()

    pl.run_scoped(outer,
                  pltpu.VMEM((BIN_PAD,), jnp.int32),
                  pltpu.VMEM((SCH,), jnp.int32),
                  pltpu.VMEM((SCH,), jnp.int32),
                  pltpu.VMEM((SCH,), jnp.int32),
                  pltpu.VMEM((SCH,), jnp.int32),
                  pltpu.VMEM((16,), jnp.int32),
                  pltpu.VMEM((SCH, HC), jnp.float32),
                  pltpu.VMEM((SCH, HC), jnp.float32),
                  pltpu.VMEM((16, HC), jnp.float32),
                  pltpu.SemaphoreType.DMA,
                  pltpu.SemaphoreType.DMA,
                  pltpu.SemaphoreType.DMA,
                  pltpu.SemaphoreType.DMA)


_SC_MESH = plsc.VectorSubcoreMesh(core_axis_name="c", subcore_axis_name="s")
_SC_PARAMS = pltpu.CompilerParams(needs_layout_passes=False)

_gather_call = functools.partial(
    pl.kernel, mesh=_SC_MESH, compiler_params=_SC_PARAMS,
    out_type=[
        jax.ShapeDtypeStruct((E, (HC + D3) // 2), jnp.int32),
        jax.ShapeDtypeStruct((NW, BIN_PAD), jnp.int32),
        jax.ShapeDtypeStruct((NW, 16), jnp.int32),
    ])(_gather_bin_kernel)

_scatter_call = functools.partial(
    pl.kernel, mesh=_SC_MESH, compiler_params=_SC_PARAMS,
    out_type=[jax.ShapeDtypeStruct((N, HC), jnp.float32)] * 4,
    scratch_types=[pltpu.VMEM_SHARED((ACC_ROWS, HC), jnp.float32)],
    )(_scatter_kernel)

_EB = 2000


def _edge_mlp(g, rbf, ev0, ev1, ev2, W1, b1, W2, b2, Wr, br):
    grid = (E // _EB,)
    row = lambda m: (m, 0)
    fixed = lambda m: (0, 0)
    return pl.pallas_call(
        _edge_mlp_kernel,
        grid=grid,
        in_specs=[
            pl.BlockSpec((_EB, (HC + D3) // 2), row),
            pl.BlockSpec((_EB, NRBF), row),
            pl.BlockSpec((_EB, 1), row),
            pl.BlockSpec((_EB, 1), row),
            pl.BlockSpec((_EB, 1), row),
            pl.BlockSpec((HC // 2, HC), fixed),
            pl.BlockSpec((1, HC // 2), fixed),
            pl.BlockSpec((D3, HC // 2), fixed),
            pl.BlockSpec((1, D3), fixed),
            pl.BlockSpec((D3, NRBF), fixed),
            pl.BlockSpec((1, D3), fixed),
        ],
        out_specs=[pl.BlockSpec((_EB, HC), row)] * 4,
        out_shape=[jax.ShapeDtypeStruct((E, HC), jnp.float32)] * 4,
    )(g, rbf, ev0, ev1, ev2, W1, b1, W2, b2, Wr, br)


def kernel(x, vec, edge_rbf, edge_vector, W1, b1, W2, b2, Wr, br, edge_index):
    vec2 = vec.reshape(N, D3)
    tbl = _pack(x, vec2)
    jj = edge_index[0]
    ii = edge_index[1]
    g, bpk, bcnt = _gather_call(tbl, jj, ii)
    mv0, mv1, mv2, mx = _edge_mlp(g, edge_rbf,
                                  edge_vector[:, 0:1], edge_vector[:, 1:2],
                                  edge_vector[:, 2:3],
                                  W1, b1.reshape(1, -1), W2, b2.reshape(1, -1),
                                  Wr, br.reshape(1, -1))
    dv0, dv1, dv2c, dx = _scatter_call(mv0, mv1, mv2, mx, bpk, bcnt)
    d_vec = jnp.stack([dv0, dv1, dv2c], axis=1)
    return (dx, d_vec)
